# Initial kernel scaffold; baseline (speedup 1.0000x reference)
import jax
import jax.numpy as jnp
from jax.experimental import pallas as pl


def kernel(A):
    raise NotImplementedError("write your pallas kernel here")


# speedup vs baseline: 8.6818x; 8.6818x over previous
import functools

import jax
import jax.numpy as jnp
from jax.experimental import pallas as pl

K = 20
ROWS_PER_BLOCK = 256


def _pass1_body(a_ref, n_ref, t_ref, dinv_ref):
    a = jnp.maximum(a_ref[0], 0.0)
    x = a + n_ref[0]
    r = a.shape[0]
    t = jnp.full((r, 1), jnp.inf, dtype=jnp.float32)
    for _ in range(K):
        t = jnp.max(jnp.where(x < t, x, -jnp.inf), axis=1, keepdims=True)
    mask = x >= t
    d = 1.0 + jnp.sum(jnp.where(mask, a, 0.0), axis=1, keepdims=True)
    t_ref[0] = jnp.broadcast_to(t, (r, 128))
    dinv_ref[0] = jnp.broadcast_to(jax.lax.rsqrt(d), (r, 128))


def _pass2_body(a_ref, n_ref, t_ref, dr_ref, dc_ref, o_ref, *, rows):
    a = jnp.maximum(a_ref[0], 0.0)
    x = a + n_ref[0]
    t = t_ref[0, :, 0:1]
    dr = dr_ref[0, :, 0:1]
    dc = dc_ref[0]
    n = a.shape[1]
    rb = pl.program_id(1)
    row_ids = rb * rows + jax.lax.broadcasted_iota(jnp.int32, (rows, n), 0)
    col_ids = jax.lax.broadcasted_iota(jnp.int32, (rows, n), 1)
    eye = jnp.where(row_ids == col_ids, 1.0, 0.0)
    m = jnp.where(x >= t, a, 0.0) + eye
    o_ref[0] = m * dr * dc


def kernel(A):
    B, N, N2 = A.shape
    assert N == N2
    R = min(ROWS_PER_BLOCK, N)
    nblk = N // R

    noise = jax.random.uniform(jax.random.key(42), A.shape, dtype=A.dtype) * 0.0001

    blk_a = pl.BlockSpec((1, R, N), lambda b, i: (b, i, 0))
    blk_t = pl.BlockSpec((1, R, 128), lambda b, i: (b, i, 0))

    t3, dinv3 = pl.pallas_call(
        _pass1_body,
        grid=(B, nblk),
        in_specs=[blk_a, blk_a],
        out_specs=[blk_t, blk_t],
        out_shape=[
            jax.ShapeDtypeStruct((B, N, 128), jnp.float32),
            jax.ShapeDtypeStruct((B, N, 128), jnp.float32),
        ],
    )(A, noise)

    dinv_col = dinv3[:, :, 0][:, None, :]

    out = pl.pallas_call(
        functools.partial(_pass2_body, rows=R),
        grid=(B, nblk),
        in_specs=[
            blk_a,
            blk_a,
            blk_t,
            blk_t,
            pl.BlockSpec((1, 1, N), lambda b, i: (b, 0, 0)),
        ],
        out_specs=blk_a,
        out_shape=jax.ShapeDtypeStruct((B, N, N), jnp.float32),
    )(A, noise, t3, dinv3, dinv_col)

    return out
